# Initial kernel scaffold; baseline (speedup 1.0000x reference)
import jax
import jax.numpy as jnp
from jax.experimental import pallas as pl


def kernel(x, node_emb_W, node_emb_b, grp_emb, rel_emb_tab, time_emb, rel_t_W, rel_t_b, fc_W, fc_b, pred_W, pred_b, node_ent, edge_index, edge_type, edge_ts, src, dst, q_rel, q_ts, ptr):
    raise NotImplementedError("write your pallas kernel here")


# speedup vs baseline: 9.0839x; 9.0839x over previous
import jax
import jax.numpy as jnp
from jax import lax
from jax.experimental import pallas as pl
from jax.experimental.pallas import tpu as pltpu
from jax.experimental.pallas import tpu_sc as plsc

N = 10000
E = 160000
BQ8 = 8
FEAT = 128
H = 64
H2 = 128
NUM_REL = 200
TSP = 368
RT_ROWS = NUM_REL * TSP

NP = 10240
NC = 2
NS = 16
SEG = 5120
SROWS = 5184
RPT = SROWS // NS
EPT = E // NS
CLCAP = EPT + 80
BQ = 64
LSCAT = 5176
LGATH = 5180
GPAD = 10008
LISTCAP = E + NS * BQ

_f32 = jnp.float32
_i32 = jnp.int32


def _lrelu(v):
    return jnp.where(v > 0, v, 0.2 * v)


def _sds(shape, dtype):
    return jax.ShapeDtypeStruct(shape, dtype)


def _k_node(x_p, g, wnT, bn8, f1T, f3T):
    blk = 1024

    def body(x_ref, g_ref, wnT_ref, bn_ref, f1_ref, f3_ref,
             h0_ref, p1_ref, p3_ref):
        u = _lrelu(jnp.dot(x_ref[...], wnT_ref[...],
                           preferred_element_type=_f32) + bn_ref[0:1, :])
        h0 = jnp.concatenate([u, g_ref[...]], axis=1)
        h0_ref[...] = h0
        p1_ref[...] = jnp.dot(h0, f1_ref[...], preferred_element_type=_f32)
        p3_ref[...] = jnp.dot(h0, f3_ref[...], preferred_element_type=_f32)

    return pl.pallas_call(
        body,
        grid=(NP // blk,),
        in_specs=[
            pl.BlockSpec((blk, FEAT), lambda i: (i, 0)),
            pl.BlockSpec((blk, H), lambda i: (i, 0)),
            pl.BlockSpec((FEAT, H), lambda i: (0, 0)),
            pl.BlockSpec((8, H), lambda i: (0, 0)),
            pl.BlockSpec((H2, H2), lambda i: (0, 0)),
            pl.BlockSpec((H2, H2), lambda i: (0, 0)),
        ],
        out_specs=[
            pl.BlockSpec((blk, H2), lambda i: (i, 0)),
            pl.BlockSpec((blk, H2), lambda i: (i, 0)),
            pl.BlockSpec((blk, H2), lambda i: (i, 0)),
        ],
        out_shape=[
            _sds((NP, H2), _f32),
            _sds((NP, H2), _f32),
            _sds((NP, H2), _f32),
        ],
    )(x_p, g, wnT, bn8, f1T, f3T)


def _k_ab(rel_emb_tab, time_p, wr1T, wr2T):
    def body(rel_ref, time_ref, w1_ref, w2_ref, a_ref, bt_ref):
        a_ref[...] = jnp.dot(rel_ref[...], w1_ref[...],
                             preferred_element_type=_f32)
        bt_ref[...] = jnp.dot(time_ref[...], w2_ref[...],
                              preferred_element_type=_f32)

    return pl.pallas_call(
        body,
        out_shape=[_sds((NUM_REL, H2), _f32), _sds((TSP, H2), _f32)],
    )(rel_emb_tab, time_p, wr1T, wr2T)


def _k_rtt(a, bt, brt8, f2T, fcb8):
    def body(a_ref, bt_ref, brt_ref, f2_ref, fcb_ref, out_ref):
        btv = bt_ref[...] + brt_ref[0:1, :]
        for j in range(8):
            rt = _lrelu(a_ref[j:j + 1, :] + btv)
            out_ref[pl.ds(j * TSP, TSP), :] = (
                jnp.dot(rt, f2_ref[...], preferred_element_type=_f32)
                + fcb_ref[0:1, :])

    return pl.pallas_call(
        body,
        grid=(NUM_REL // 8,),
        in_specs=[
            pl.BlockSpec((8, H2), lambda i: (i, 0)),
            pl.BlockSpec((TSP, H2), lambda i: (0, 0)),
            pl.BlockSpec((8, H2), lambda i: (0, 0)),
            pl.BlockSpec((H2, H2), lambda i: (0, 0)),
            pl.BlockSpec((8, H2), lambda i: (0, 0)),
        ],
        out_specs=pl.BlockSpec((8 * TSP, H2), lambda i: (i, 0)),
        out_shape=_sds((RT_ROWS, H2), _f32),
    )(a, bt, brt8, f2T, fcb8)


def _k_delta(newh, f13T):
    blk = 1024

    def body(nh_ref, w_ref, d_ref):
        d_ref[...] = jnp.dot(nh_ref[...], w_ref[...],
                             preferred_element_type=_f32)

    return pl.pallas_call(
        body,
        grid=(NP // blk,),
        in_specs=[
            pl.BlockSpec((blk, H2), lambda i: (i, 0)),
            pl.BlockSpec((H2, 2 * H2), lambda i: (0, 0)),
        ],
        out_specs=pl.BlockSpec((blk, 2 * H2), lambda i: (i, 0)),
        out_shape=_sds((NP, 2 * H2), _f32),
    )(newh, f13T)


def _k_pred(hq, qrt, pw8, pb8):
    def body(hq_ref, qrt_ref, pw_ref, pb_ref, out_ref):
        hqv = hq_ref[...]
        w = pw_ref[0:1, :]
        z = (jnp.sum(hqv[0:8, :] * w[:, 0:H2], axis=1, keepdims=True)
             + jnp.sum(qrt_ref[...] * w[:, H2:2 * H2], axis=1, keepdims=True)
             + jnp.sum(hqv[8:16, :] * w[:, 2 * H2:], axis=1, keepdims=True)
             + pb_ref[0:1, 0:1])
        out_ref[...] = jnp.broadcast_to(jax.nn.sigmoid(z), (BQ8, H2))

    return pl.pallas_call(
        body,
        out_shape=_sds((BQ8, H2), _f32),
    )(hq, qrt, pw8, pb8)


_MESH = plsc.VectorSubcoreMesh(core_axis_name="c", subcore_axis_name="s",
                               num_cores=NC, num_subcores=NS)
_SC_PARAMS = pltpu.CompilerParams(needs_layout_passes=False,
                                  use_tc_tiling_on_sc=False)
_GPT = NP // (NC * NS)


def _k0_body(grp_ref, ent_ref, tq_ref, g_ref, forb_ref,
             v_ent, v_g, v_forb, v_tq, sem):
    sc = lax.axis_index("c")
    tid = lax.axis_index("s")
    base = (sc * NS + tid) * _GPT
    pltpu.sync_copy(ent_ref.at[pl.ds(base, _GPT)], v_ent)
    pltpu.sync_copy(tq_ref, v_tq)
    pltpu.async_copy(grp_ref.at[v_ent], v_g, sem).wait()

    @pl.loop(0, _GPT // 16)
    def _(i):
        v = base + i * 16 + lax.iota(_i32, 16)
        nb = jnp.zeros((16,), _i32)
        for b in range(8):
            thrb = plsc.load_gather(v_tq, [jnp.full((16,), b, _i32)])
            nb += (v >= thrb).astype(_i32)
        nb = jnp.minimum(nb, 7)
        v_forb[pl.ds(i * 16, 16)] = plsc.load_gather(v_tq, [nb + 8])

    pltpu.sync_copy(v_g, g_ref.at[pl.ds(base, _GPT)])
    pltpu.sync_copy(v_forb, forb_ref.at[pl.ds(base, _GPT)])


def _k0(grp_emb, ent_p, tq):
    return pl.kernel(
        _k0_body,
        out_type=(_sds((NP, H), _f32), _sds((NP,), _i32)),
        mesh=_MESH,
        compiler_params=_SC_PARAMS,
        scratch_types=[
            pltpu.VMEM((_GPT,), _i32),
            pltpu.VMEM((_GPT, H), _f32),
            pltpu.VMEM((_GPT,), _i32),
            pltpu.VMEM((16,), _i32),
            pltpu.SemaphoreType.DMA,
        ],
    )(grp_emb, ent_p, tq)


def _apply_body(fin_ids, fin_cnt, dprev, p1, p3, out_dummy,
                v_ids64, v_b1, v_b3, v_bm, v_32i, sem):
    sc = lax.axis_index("c")
    tid = lax.axis_index("s")
    w = sc * NS + tid
    pltpu.sync_copy(fin_cnt, v_32i)
    cnt_a = v_32i[pl.ds(0, 16)]
    cnt_b = v_32i[pl.ds(16, 16)]
    for seg in range(2):
        cc = cnt_a[0] if seg == 0 else cnt_b[0]

        @pl.loop(w, cc // BQ, step=NC * NS)
        def _(gi):
            slot = pl.multiple_of(seg * SEG + gi * BQ, BQ)
            pltpu.sync_copy(fin_ids.at[pl.ds(slot, BQ)], v_ids64)
            pltpu.async_copy(p1.at[v_ids64], v_b1, sem).wait()
            pltpu.async_copy(p3.at[v_ids64], v_b3, sem).wait()
            pltpu.sync_copy(dprev.at[pl.ds(slot, BQ), pl.ds(0, H2)], v_bm)

            @pl.loop(0, BQ)
            def _(r):
                for c in range(8):
                    cs = pl.ds(c * 16, 16)
                    v_b1[r, cs] = v_b1[r, cs] + v_bm[r, cs]

            pltpu.sync_copy(dprev.at[pl.ds(slot, BQ), pl.ds(H2, H2)], v_bm)

            @pl.loop(0, BQ)
            def _(r):
                for c in range(8):
                    cs = pl.ds(c * 16, 16)
                    v_b3[r, cs] = v_b3[r, cs] + v_bm[r, cs]

            pltpu.async_copy(v_b1, p1.at[v_ids64], sem).wait()
            pltpu.async_copy(v_b3, p3.at[v_ids64], sem).wait()

    @pl.when((tid == 0) & (sc == 0))
    def _():
        v_32i[pl.ds(0, 16)] = jnp.zeros((16,), _i32)
        pltpu.sync_copy(v_32i.at[pl.ds(0, 16)], out_dummy)


_k_apply = pl.kernel(
    _apply_body,
    out_type=(_sds((16,), _i32),),
    mesh=_MESH,
    compiler_params=_SC_PARAMS,
    scratch_types=[
        pltpu.VMEM((BQ,), _i32),
        pltpu.VMEM((BQ, H2), _f32),
        pltpu.VMEM((BQ, H2), _f32),
        pltpu.VMEM((BQ, H2), _f32),
        pltpu.VMEM((32,), _i32),
        pltpu.SemaphoreType.DMA,
    ],
)


def _make_round(rnd):
    def body(esrc, edst, ertx, forb, fin_ids, fin_cnt, rtt, h0,
             qids, a_tab, bt_tab, brt_vec, qq,
             p1, p3, hq,
             rout_ids, rout_newh, rout_cnt, qrt_out, o_els, o_eld,
             s_sums, s_cnt,
             v_tbl, v_s0, v_s1, v_s2, v_cls, v_cld,
             v_b1, v_b3, v_brt, v_bm, v_bones, v_z16, v_c64,
             v_bsrc, v_bdst, v_brtx, v_ids64, v_idsg,
             v_cnt, v_row, v_row0, v_row2,
             v_32i, v_16q, v_16f, m_ctr, sem):
        sc = lax.axis_index("c")
        tid = lax.axis_index("s")
        zb16 = jnp.zeros((16,), _f32)

        @pl.when(tid == 0)
        def _():
            m_ctr[0] = 0
            m_ctr[1] = 0
            m_ctr[2] = 0

        @pl.loop(0, BQ)
        def _(r):
            v_bones[r, :] = jnp.ones((16,), _f32)
            v_z16[r, :] = zb16
            for c in range(8):
                v_bm[r, pl.ds(c * 16, 16)] = zb16

        rt0 = tid * RPT
        for j, rows in enumerate((64, 64, 64, 64, 64, 4)):
            off = rt0 + j * 64
            pltpu.sync_copy(v_bm.at[pl.ds(0, rows), :],
                            s_sums.at[pl.ds(off, rows), :])
            pltpu.sync_copy(v_z16.at[pl.ds(0, rows), :],
                            s_cnt.at[pl.ds(off, rows), :])

        pltpu.sync_copy(fin_cnt, v_32i)
        cnt_a = v_32i[pl.ds(0, 16)]
        cnt_b = v_32i[pl.ds(16, 16)]
        if rnd < 3:
            for p in range(NP // 2048):
                pltpu.sync_copy(forb.at[pl.ds(p * 2048, 2048)], v_s0)

                @pl.loop(0, 128)
                def _(i):
                    v_tbl[pl.ds(p * 2048 + i * 16, 16)] = (
                        v_s0[pl.ds(i * 16, 16)] << 1)

            for seg in range(2):
                cc = cnt_a[0] if seg == 0 else cnt_b[0]

                @pl.loop(0, cc // BQ)
                def _(gi):
                    off = pl.multiple_of(seg * SEG + gi * BQ, BQ)
                    pltpu.sync_copy(fin_ids.at[pl.ds(off, BQ)], v_ids64)
                    for j in range(4):
                        idv = v_ids64[pl.ds(j * 16, 16)]
                        cur = plsc.load_gather(v_tbl, [idv])
                        plsc.store_scatter(v_tbl, [idv], cur | 1)
        else:
            @pl.loop(0, NP // 16)
            def _(i):
                v_tbl[pl.ds(i * 16, 16)] = jnp.zeros((16,), _i32)

            pltpu.sync_copy(qids, v_16q)
            qv = v_16q[...]
            plsc.store_scatter(v_tbl, [qv], jnp.ones((16,), _i32),
                               mask=lax.iota(_i32, 16) >= 8)

        plsc.subcore_barrier()

        ECH = 2000

        def scan_body(i, lpos):
            sv = v_s0[pl.ds(i * 16, 16)]
            dv = v_s1[pl.ds(i * 16, 16)]
            rv = v_s2[pl.ds(i * 16, 16)]
            dloc = dv - sc * SEG
            inr = (dloc >= 0) & (dloc < SEG)
            if rnd < 3:
                g = plsc.load_gather(v_tbl, [sv])
                act = ((g & 1) == 1) & (dv != (g >> 1)) & inr
            else:
                g = plsc.load_gather(v_tbl, [dv])
                act = (g == 1) & inr
            plsc.store_compressed(v_cls.at[pl.ds(lpos, 16)], sv, mask=act)
            plsc.store_compressed(v_cld.at[pl.ds(lpos, 16)],
                                  (dloc << 17) | rv, mask=act)
            c = jnp.max(plsc.all_reduce_population_count(act))
            return lpos + c

        lpos = jnp.int32(0)
        for p in range(EPT // ECH):
            ebase = pl.multiple_of(tid * EPT + p * ECH, 16)
            pltpu.sync_copy(esrc.at[pl.ds(ebase, ECH)],
                            v_s0.at[pl.ds(0, ECH)])
            pltpu.sync_copy(edst.at[pl.ds(ebase, ECH)],
                            v_s1.at[pl.ds(0, ECH)])
            pltpu.sync_copy(ertx.at[pl.ds(ebase, ECH)],
                            v_s2.at[pl.ds(0, ECH)])
            lpos = lax.fori_loop(0, ECH // 16, scan_body, lpos)
        ones16b = jnp.ones((16,), jnp.bool_)
        for j in range(4):
            plsc.store_compressed(v_cls.at[pl.ds(lpos + j * 16, 16)],
                                  jnp.zeros((16,), _i32), mask=ones16b)
            plsc.store_compressed(v_cld.at[pl.ds(lpos + j * 16, 16)],
                                  jnp.full((16,), LSCAT << 17, _i32),
                                  mask=ones16b)
        lpos_p = ((lpos + BQ - 1) // BQ) * BQ
        lbase = plsc.fetch_and_add(m_ctr.at[0], lpos_p, subcore_id=0)

        @pl.loop(0, lpos_p // BQ)
        def _(j):
            src_o = pl.multiple_of(j * BQ, BQ)
            dst_o = pl.multiple_of(lbase + j * BQ, BQ)
            pltpu.sync_copy(v_cls.at[pl.ds(src_o, BQ)],
                            o_els.at[sc, pl.ds(dst_o, BQ)])
            pltpu.sync_copy(v_cld.at[pl.ds(src_o, BQ)],
                            o_eld.at[sc, pl.ds(dst_o, BQ)])

        plsc.subcore_barrier()

        total = plsc.fetch_and_add(m_ctr.at[0], 0, subcore_id=0)

        def msg_batch(wb):
            wbo = pl.multiple_of(wb, BQ)
            pltpu.sync_copy(o_els.at[sc, pl.ds(wbo, BQ)], v_bsrc)
            pltpu.sync_copy(o_eld.at[sc, pl.ds(wbo, BQ)],
                            v_cld.at[pl.ds(0, BQ)])
            for j in range(4):
                pk = v_cld[pl.ds(j * 16, 16)]
                dv = pk >> 17
                v_bdst[pl.ds(j * 16, 16)] = dv
                v_brtx[pl.ds(j * 16, 16)] = pk & 0x1FFFF
                v_idsg[pl.ds(j * 16, 16)] = jnp.minimum(
                    dv + sc * SEG, NP - 8)
            pltpu.async_copy(p1.at[v_bsrc], v_b1, sem).wait()
            pltpu.async_copy(p3.at[v_idsg], v_b3, sem).wait()
            pltpu.async_copy(rtt.at[v_brtx], v_brt, sem).wait()

            @pl.loop(0, BQ)
            def _(r):
                for c in range(8):
                    cs = pl.ds(c * 16, 16)
                    m = v_b1[r, cs] + v_b3[r, cs] + v_brt[r, cs]
                    v_bm[r, cs] = jnp.where(m > 0, m, 0.2 * m)

            pltpu.async_copy(v_bm, s_sums.at[v_bdst], sem, add=True).wait()
            pltpu.async_copy(v_bones, s_cnt.at[v_bdst], sem, add=True).wait()
            return plsc.fetch_and_add(m_ctr.at[1], BQ, subcore_id=0)

        wb0 = plsc.fetch_and_add(m_ctr.at[1], BQ, subcore_id=0)
        lax.while_loop(lambda wb: wb < total, msg_batch, wb0)

        plsc.subcore_barrier()

        @pl.when(tid == 0)
        def _():
            pltpu.sync_copy(qids, v_16q)
            qv16 = v_16q[...]
            for j in range(16):
                v = qv16[j]
                loc = v - sc * SEG
                owned = (loc >= 0) & (loc < SEG)

                @pl.when(owned)
                def _():
                    lc = jnp.clip(loc, 0, SROWS - 1)
                    pltpu.sync_copy(s_cnt.at[lc], v_16f)
                    pltpu.sync_copy(s_sums.at[lc], v_row)
                    if rnd == 1:
                        pltpu.sync_copy(h0.at[v], v_row0)
                    else:
                        pltpu.sync_copy(hq.at[j], v_row0)
                    cvec = v_16f[...]
                    scale = jnp.where(cvec > 0, 1.0, 0.0) / jnp.maximum(
                        cvec, 1.0)
                    for c in range(8):
                        cs = pl.ds(c * 16, 16)
                        v_row2[cs] = v_row0[cs] + v_row[cs] * scale
                    pltpu.sync_copy(v_row2, hq.at[j])

        if rnd < 3:
            pltpu.sync_copy(s_cnt.at[pl.ds(rt0, RPT), :], v_cnt)
            zeros16 = jnp.zeros((16,), _i32)

            def recv_body(i, pos):
                lrow = i * 16 + lax.iota(_i32, 16)
                safe = jnp.minimum(lrow, RPT - 1)
                cv = plsc.load_gather(v_cnt, [safe, zeros16])
                loc = rt0 + lrow
                take = (cv > 0) & (lrow < RPT) & (loc < SEG)
                plsc.store_compressed(v_cls.at[pl.ds(pos, 16)], loc,
                                      mask=take)
                return pos + jnp.max(
                    plsc.all_reduce_population_count(take))

            pos = lax.fori_loop(0, (RPT + 15) // 16, recv_body,
                                jnp.int32(0))
            ones16c = jnp.ones((16,), jnp.bool_)
            for j in range(4):
                plsc.store_compressed(v_cls.at[pl.ds(pos + j * 16, 16)],
                                      jnp.full((16,), LGATH, _i32),
                                      mask=ones16c)
            pos_p = ((pos + BQ - 1) // BQ) * BQ
            sbase = plsc.fetch_and_add(m_ctr.at[2], pos_p, subcore_id=0)

            @pl.loop(0, pos_p // BQ)
            def _(jb):
                jbo = pl.multiple_of(jb * BQ, BQ)
                for j in range(4):
                    v_ids64[pl.ds(j * 16, 16)] = v_cls[
                        pl.ds(jbo + j * 16, 16)]
                pltpu.async_copy(s_sums.at[v_ids64], v_b1, sem).wait()
                pltpu.async_copy(s_cnt.at[v_ids64], v_c64, sem).wait()

                @pl.loop(0, BQ)
                def _(r):
                    cvec = v_c64[r]
                    inv = 1.0 / jnp.maximum(cvec, 1.0)
                    for c in range(8):
                        cs = pl.ds(c * 16, 16)
                        v_bm[r, cs] = v_b1[r, cs] * inv

                for j in range(4):
                    lv = v_ids64[pl.ds(j * 16, 16)]
                    v_idsg[pl.ds(j * 16, 16)] = jnp.where(
                        lv >= SEG, GPAD, lv + sc * SEG)
                out0 = pl.multiple_of(sc * SEG + sbase + jb * BQ, BQ)
                pltpu.sync_copy(v_idsg, rout_ids.at[pl.ds(out0, BQ)])
                pltpu.sync_copy(v_bm, rout_newh.at[pl.ds(out0, BQ), :])

            plsc.subcore_barrier()

            @pl.when(tid == 0)
            def _():
                tot = plsc.fetch_and_add(m_ctr.at[2], 0, subcore_id=0)
                v_32i[pl.ds(0, 16)] = jnp.where(lax.iota(_i32, 16) == 0,
                                                tot, 0)
                cnt_o = pl.multiple_of(sc * 16, 16)
                pltpu.sync_copy(v_32i.at[pl.ds(0, 16)],
                                rout_cnt.at[pl.ds(cnt_o, 16)])

        if rnd == 3:
            @pl.when((tid == 0) & (sc == 0))
            def _():
                pltpu.sync_copy(qq, v_16q)
                qv16 = v_16q[...]
                pltpu.sync_copy(brt_vec, v_row2)
                for j in range(8):
                    pltpu.sync_copy(a_tab.at[qv16[j]], v_row)
                    pltpu.sync_copy(bt_tab.at[qv16[8 + j]], v_row0)
                    for c in range(8):
                        cs = pl.ds(c * 16, 16)
                        s = v_row[cs] + v_row0[cs] + v_row2[cs]
                        v_row[cs] = jnp.where(s > 0, s, 0.2 * s)
                    pltpu.sync_copy(v_row, qrt_out.at[j])

    return pl.kernel(
        body,
        out_type=(
            _sds((NP,), _i32),
            _sds((NP, H2), _f32),
            _sds((NC * 16,), _i32),
            _sds((BQ8, H2), _f32),
            _sds((NC, LISTCAP), _i32),
            _sds((NC, LISTCAP), _i32),
        ),
        mesh=_MESH,
        compiler_params=_SC_PARAMS,
        scratch_types=[
            pltpu.VMEM_SHARED((SROWS, H2), _f32),
            pltpu.VMEM_SHARED((SROWS, 16), _f32),
            pltpu.VMEM((NP,), _i32),
            pltpu.VMEM((2048,), _i32),
            pltpu.VMEM((2048,), _i32),
            pltpu.VMEM((2048,), _i32),
            pltpu.VMEM((CLCAP,), _i32),
            pltpu.VMEM((CLCAP,), _i32),
            pltpu.VMEM((BQ, H2), _f32),
            pltpu.VMEM((BQ, H2), _f32),
            pltpu.VMEM((BQ, H2), _f32),
            pltpu.VMEM((BQ, H2), _f32),
            pltpu.VMEM((BQ, 16), _f32),
            pltpu.VMEM((BQ, 16), _f32),
            pltpu.VMEM((BQ, 16), _f32),
            pltpu.VMEM((BQ,), _i32),
            pltpu.VMEM((BQ,), _i32),
            pltpu.VMEM((BQ,), _i32),
            pltpu.VMEM((BQ,), _i32),
            pltpu.VMEM((BQ,), _i32),
            pltpu.VMEM((RPT, 16), _f32),
            pltpu.VMEM((H2,), _f32),
            pltpu.VMEM((H2,), _f32),
            pltpu.VMEM((H2,), _f32),
            pltpu.VMEM((32,), _i32),
            pltpu.VMEM((16,), _i32),
            pltpu.VMEM((16,), _f32),
            pltpu.SMEM((8,), _i32),
            pltpu.SemaphoreType.DMA,
        ],
    )


_round1 = _make_round(1)
_round2 = _make_round(2)
_round3 = _make_round(3)


def kernel(x, node_emb_W, node_emb_b, grp_emb, rel_emb_tab, time_emb,
           rel_t_W, rel_t_b, fc_W, fc_b, pred_W, pred_b, node_ent,
           edge_index, edge_type, edge_ts, src, dst, q_rel, q_ts, ptr):
    ptr = ptr.astype(_i32)
    q_s = (src + ptr[:-1]).astype(_i32)
    q_o = (dst + ptr[:-1]).astype(_i32)
    qids = jnp.concatenate([q_s, q_o])
    tq = jnp.concatenate([ptr[1:9], q_o])
    qq = jnp.concatenate([q_rel.astype(_i32), q_ts.astype(_i32)])

    ent_p = jnp.zeros((NP,), _i32).at[:N].set(node_ent.astype(_i32))
    x_p = jnp.zeros((NP, FEAT), _f32).at[:N].set(x)
    e_src = edge_index[0].astype(_i32)
    e_dst = edge_index[1].astype(_i32)
    rtix = edge_type.astype(_i32) * TSP + edge_ts.astype(_i32)
    time_p = jnp.zeros((TSP, H), _f32).at[:time_emb.shape[0]].set(time_emb)

    wnT = node_emb_W.T
    bn8 = jnp.broadcast_to(node_emb_b[None, :], (8, H))
    f1T = fc_W[:, 0:H2].T
    f2T = fc_W[:, H2:2 * H2].T
    f3T = fc_W[:, 2 * H2:].T
    f13T = jnp.concatenate([f1T, f3T], axis=1)
    wr1T = rel_t_W[:, :H].T
    wr2T = rel_t_W[:, H:].T
    brt8 = jnp.broadcast_to(rel_t_b[None, :], (8, H2))
    fcb8 = jnp.broadcast_to(fc_b[None, :], (8, H2))
    pw8 = jnp.broadcast_to(pred_W.reshape(1, 3 * H2), (8, 3 * H2))
    pb8 = jnp.broadcast_to(pred_b.reshape(1, 1), (8, H2))

    g, _ = _k0(grp_emb, ent_p, tq)
    vv = jnp.arange(NP, dtype=_i32)
    nbat = jnp.minimum(
        sum((vv >= ptr[j]).astype(_i32) for j in range(1, 9)), 7)
    forb = q_o[nbat]
    h0, p1v, p3v = _k_node(x_p, g, wnT, bn8, f1T, f3T)
    a_tab, bt_tab = _k_ab(rel_emb_tab, time_p, wr1T, wr2T)
    rtt = _k_rtt(a_tab, bt_tab, brt8, f2T, fcb8)

    p1 = jax.new_ref(p1v)
    p3 = jax.new_ref(p3v)
    hq = jax.new_ref(jnp.zeros((16, H2), _f32))

    fin_ids0 = jnp.full((NP,), GPAD, _i32).at[:8].set(q_s)
    fin_cnt0 = jnp.zeros((NC * 16,), _i32).at[0].set(BQ)

    d_tab = jnp.zeros((8, H2), _f32)
    d_vec = jnp.zeros((H2,), _f32)
    d_q = jnp.zeros((16,), _i32)
    d_forb = jnp.zeros((8,), _i32)
    d_h0 = jnp.zeros((8, H2), _f32)

    ids1, newh1, cnt1, _, _, _ = _round1(
        e_src, e_dst, rtix, forb, fin_ids0, fin_cnt0, rtt, h0,
        qids, d_tab, d_tab, d_vec, d_q, p1, p3, hq)
    dlt1 = _k_delta(newh1, f13T)
    _k_apply(ids1, cnt1, dlt1, p1, p3)
    ids2, newh2, cnt2, _, _, _ = _round2(
        e_src, e_dst, rtix, forb, ids1, cnt1, rtt, d_h0,
        qids, d_tab, d_tab, d_vec, d_q, p1, p3, hq)
    dlt2 = _k_delta(newh2, f13T)
    _k_apply(ids2, cnt2, dlt2, p1, p3)
    _, _, _, qrt, _, _ = _round3(
        e_src, e_dst, rtix, d_forb, ids2, cnt2, rtt, d_h0,
        qids, a_tab, bt_tab, rel_t_b, qq, p1, p3, hq)

    out2 = _k_pred(hq[...], qrt, pw8, pb8)
    return out2[:, 0]
